# baseline (speedup 1.0000x reference)
import functools

import jax
import jax.numpy as jnp
from jax import lax
from jax.experimental import pallas as pl
from jax.experimental.pallas import tpu as pltpu
from jax.experimental.pallas import tpu_sc as plsc

H = 8
D = 128
IND = 128
NN = 10000
EE = 160000

NC = 2
NS = 16
NW = NC * NS
EPT = 5008
EPAD = NW * EPT
STR = NN + 16
FPAD = H * STR
SPT = FPAD // NS
NCH = 40
CW = 128
VPR = CW // 16
TLE = NCH * CW
NIT2 = NCH * VPR
NBLK = 2 * H


def _tc_prologue(x1, x2, W1, as1, ad1, W2, as2, ad2, als_o, ald_o, ini_o):
    hd_iota = lax.broadcasted_iota(jnp.int32, (H * D, H), 0) // D
    h_iota = lax.broadcasted_iota(jnp.int32, (H * D, H), 1)
    B = jnp.where(hd_iota == h_iota, 1.0, 0.0)
    for g, (x, W, asv, adv) in enumerate(((x1, W1, as1, ad1),
                                          (x2, W2, as2, ad2))):
        xv = x[...]
        Wv = W[...]
        A_s = jnp.dot(Wv * asv[...], B, preferred_element_type=jnp.float32)
        A_d = jnp.dot(Wv * adv[...], B, preferred_element_type=jnp.float32)
        als = lax.dot_general(A_s, xv, (((0,), (1,)), ((), ())),
                              preferred_element_type=jnp.float32)
        ald = lax.dot_general(A_d, xv, (((0,), (1,)), ((), ())),
                              preferred_element_type=jnp.float32)
        z = als + ald
        ini_o[g] = jnp.exp(jnp.maximum(z, 0.2 * z))
        als_o[g] = als
        ald_o[g] = ald


def _run_tc_prologue(x1, x2, W1, as1f, ad1f, W2, as2f, ad2f):
    shp = jax.ShapeDtypeStruct((2, H, NN), jnp.float32)
    return pl.pallas_call(
        _tc_prologue,
        out_shape=[shp, shp, shp],
    )(x1, x2, W1, as1f, ad1f, W2, as2f, ad2f)


_SC_MESH = plsc.VectorSubcoreMesh(core_axis_name="c", subcore_axis_name="s")

_F1 = jax.ShapeDtypeStruct((FPAD,), jnp.float32)
_EB = jax.ShapeDtypeStruct((2 * H * NW, NCH, CW), jnp.float32)


def _edge_tail_init(src_t, dst_t):
    pad16 = jnp.full((16,), NN, jnp.int32)
    for k in range(EPT, TLE, 16):
        src_t[pl.ds(k, 16)] = pad16
        dst_t[pl.ds(k, 16)] = pad16


@functools.partial(
    pl.kernel,
    out_type=[_F1, _F1, _F1, _F1, _EB],
    mesh=_SC_MESH,
    compiler_params=pltpu.CompilerParams(needs_layout_passes=False),
    scratch_types=[
        pltpu.VMEM((STR,), jnp.float32),
        pltpu.VMEM((STR,), jnp.float32),
        pltpu.VMEM((STR,), jnp.float32),
        pltpu.VMEM((STR,), jnp.float32),
        pltpu.VMEM((TLE,), jnp.int32),
        pltpu.VMEM((TLE,), jnp.int32),
        pltpu.VMEM((NCH, CW), jnp.float32),
        pltpu.VMEM((NCH, CW), jnp.int32),
        pltpu.VMEM((NCH, CW), jnp.float32),
        pltpu.VMEM((NCH, CW), jnp.int32),
        pltpu.VMEM((SPT,), jnp.float32),
        pltpu.VMEM_SHARED((FPAD,), jnp.float32),
        pltpu.VMEM_SHARED((FPAD,), jnp.float32),
        pltpu.SemaphoreType.DMA,
        pltpu.SemaphoreType.DMA,
        pltpu.SemaphoreType.DMA,
    ],
)
def _sc_pass1(src1, dst1, src2, dst2, als1, ald1, als2, ald2, ini1, ini2,
              d1c0, d1c1, d2c0, d2c1, eeb,
              as_a, ad_a, as_b, ad_b, src_t, dst_t,
              vbufa, ibufa, vbufb, ibufb,
              stg, dsh0, dsh1, sem_d, sem_t, sem_e):
    c = lax.axis_index("c")
    s = lax.axis_index("s")
    w = c * NS + s

    _edge_tail_init(src_t, dst_t)

    factor = jnp.where(c == 0, 1.0, 0.0).astype(jnp.float32)
    for g in range(2):
        dsh = dsh0 if g == 0 else dsh1
        ini = ini1 if g == 0 else ini2
        pltpu.sync_copy(ini.at[pl.ds(s * SPT, SPT)], stg)

        @plsc.parallel_loop(0, SPT // 16, unroll=8)
        def _scale(i):
            stg[pl.ds(i * 16, 16)] = stg[pl.ds(i * 16, 16)] * factor
        pltpu.sync_copy(stg, dsh.at[pl.ds(s * SPT, SPT)])
    plsc.subcore_barrier()

    def tbl(idx):
        g, h = divmod(idx, H)
        a = (als1 if g == 0 else als2).at[pl.ds(h * NN, NN)]
        b = (ald1 if g == 0 else ald2).at[pl.ds(h * NN, NN)]
        t = (as_a, ad_a) if idx % 2 == 0 else (as_b, ad_b)
        return (a, t[0].at[pl.ds(0, NN)]), (b, t[1].at[pl.ds(0, NN)])

    pltpu.sync_copy(src1.at[pl.ds(w * EPT, EPT)], src_t.at[pl.ds(0, EPT)])
    pltpu.sync_copy(dst1.at[pl.ds(w * EPT, EPT)], dst_t.at[pl.ds(0, EPT)])
    for pair in tbl(0):
        pltpu.async_copy(pair[0], pair[1], sem_t)

    prev = None
    for idx in range(NBLK):
        g, h = divmod(idx, H)
        dsh = dsh0 if g == 0 else dsh1
        as_t, ad_t = (as_a, ad_a) if idx % 2 == 0 else (as_b, ad_b)
        vbuf, ibuf = (vbufa, ibufa) if idx % 2 == 0 else (vbufb, ibufb)
        if (g, h) == (1, 0):
            pltpu.sync_copy(src2.at[pl.ds(w * EPT, EPT)],
                            src_t.at[pl.ds(0, EPT)])
            pltpu.sync_copy(dst2.at[pl.ds(w * EPT, EPT)],
                            dst_t.at[pl.ds(0, EPT)])
        for pair in tbl(idx):
            pltpu.make_async_copy(pair[0], pair[1], sem_t).wait()
        if idx + 1 < NBLK:
            for pair in tbl(idx + 1):
                pltpu.async_copy(pair[0], pair[1], sem_t)
        hoff = h * STR

        @plsc.parallel_loop(0, NIT2, unroll=16)
        def _edge(i):
            off = i * 16
            s16 = src_t[pl.ds(off, 16)]
            d16 = dst_t[pl.ds(off, 16)]
            z = (plsc.load_gather(as_t, [s16])
                 + plsc.load_gather(ad_t, [d16]))
            ee = jnp.exp(jnp.maximum(z, 0.2 * z))
            vbuf[i // VPR, pl.ds((i % VPR) * 16, 16)] = ee
            ibuf[i // VPR, pl.ds((i % VPR) * 16, 16)] = d16 + hoff

        if prev is not None:
            pv, pi, pd, pblk = prev

            def _drain(r, _):
                pltpu.make_async_copy(pv.at[r], pd.at[pi.at[r]],
                                      sem_d).wait()
                return 0
            lax.fori_loop(0, NCH, _drain, 0)
            pltpu.make_async_copy(pv, eeb.at[pblk], sem_e).wait()

        def _fire(r, _):
            pltpu.async_copy(vbuf.at[r], dsh.at[ibuf.at[r]], sem_d,
                             add=True)
            return 0
        lax.fori_loop(0, NCH, _fire, 0)
        blk = idx * NW + w
        pltpu.async_copy(vbuf, eeb.at[blk], sem_e)
        prev = (vbuf, ibuf, dsh, blk)

    pv, pi, pd, pblk = prev

    def _drain_last(r, _):
        pltpu.make_async_copy(pv.at[r], pd.at[pi.at[r]], sem_d).wait()
        return 0
    lax.fori_loop(0, NCH, _drain_last, 0)
    pltpu.make_async_copy(pv, eeb.at[pblk], sem_e).wait()
    plsc.subcore_barrier()

    for g in range(2):
        dsh = dsh0 if g == 0 else dsh1
        out_c0 = d1c0 if g == 0 else d2c0
        out_c1 = d1c1 if g == 0 else d2c1
        pltpu.sync_copy(dsh.at[pl.ds(s * SPT, SPT)], stg)

        @pl.when(c == 0)
        def _():
            pltpu.sync_copy(stg, out_c0.at[pl.ds(s * SPT, SPT)])

        @pl.when(c == 1)
        def _():
            pltpu.sync_copy(stg, out_c1.at[pl.ds(s * SPT, SPT)])


@functools.partial(
    pl.kernel,
    out_type=[_F1, _F1, _F1, _F1],
    mesh=_SC_MESH,
    compiler_params=pltpu.CompilerParams(needs_layout_passes=False),
    scratch_types=[
        pltpu.VMEM((STR,), jnp.float32),
        pltpu.VMEM((STR,), jnp.float32),
        pltpu.VMEM((TLE,), jnp.int32),
        pltpu.VMEM((TLE,), jnp.int32),
        pltpu.VMEM((NCH, CW), jnp.float32),
        pltpu.VMEM((NCH, CW), jnp.int32),
        pltpu.VMEM((NCH, CW), jnp.float32),
        pltpu.VMEM((NCH, CW), jnp.int32),
        pltpu.VMEM((SPT,), jnp.float32),
        pltpu.VMEM((SPT,), jnp.float32),
        pltpu.VMEM((SPT,), jnp.float32),
        pltpu.VMEM_SHARED((FPAD,), jnp.float32),
        pltpu.VMEM_SHARED((FPAD,), jnp.float32),
        pltpu.VMEM_SHARED((FPAD,), jnp.float32),
        pltpu.VMEM_SHARED((FPAD,), jnp.float32),
        pltpu.SemaphoreType.DMA,
        pltpu.SemaphoreType.DMA,
        pltpu.SemaphoreType.DMA,
    ],
)
def _sc_pass2(src1, dst1, src2, dst2, ini1, ini2,
              d1c0, d1c1, d2c0, d2c1, eeb,
              w1c0, w1c1, w2c0, w2c1,
              di_a, di_b, src_t, dst_t,
              vbufa, ibufa, vbufb, ibufb,
              stg0, stg1, stg2, ish0, ish1, wsh0, wsh1,
              sem_d, sem_t, sem_e):
    c = lax.axis_index("c")
    s = lax.axis_index("s")
    w = c * NS + s

    _edge_tail_init(src_t, dst_t)

    factor = jnp.where(c == 0, 1.0, 0.0).astype(jnp.float32)
    for g in range(2):
        ish = ish0 if g == 0 else ish1
        wsh = wsh0 if g == 0 else wsh1
        dp0 = d1c0 if g == 0 else d2c0
        dp1 = d1c1 if g == 0 else d2c1
        ini = ini1 if g == 0 else ini2
        pltpu.sync_copy(dp0.at[pl.ds(s * SPT, SPT)], stg0)
        pltpu.sync_copy(dp1.at[pl.ds(s * SPT, SPT)], stg1)
        pltpu.sync_copy(ini.at[pl.ds(s * SPT, SPT)], stg2)

        @plsc.parallel_loop(0, SPT // 16, unroll=8)
        def _inv(i):
            dv = 1.0 / (stg0[pl.ds(i * 16, 16)] + stg1[pl.ds(i * 16, 16)])
            stg0[pl.ds(i * 16, 16)] = dv
            stg1[pl.ds(i * 16, 16)] = stg2[pl.ds(i * 16, 16)] * dv * factor
        pltpu.sync_copy(stg0, ish.at[pl.ds(s * SPT, SPT)])
        pltpu.sync_copy(stg1, wsh.at[pl.ds(s * SPT, SPT)])
    plsc.subcore_barrier()

    def dtbl(idx):
        g, h = divmod(idx, H)
        ish = ish0 if g == 0 else ish1
        t = di_a if idx % 2 == 0 else di_b
        return ish.at[pl.ds(h * STR, STR)], t

    def ebl(idx):
        vbuf = vbufa if idx % 2 == 0 else vbufb
        return eeb.at[idx * NW + w], vbuf

    pltpu.sync_copy(src1.at[pl.ds(w * EPT, EPT)], src_t.at[pl.ds(0, EPT)])
    pltpu.sync_copy(dst1.at[pl.ds(w * EPT, EPT)], dst_t.at[pl.ds(0, EPT)])
    a, b = dtbl(0)
    pltpu.async_copy(a, b, sem_t)
    a, b = ebl(0)
    pltpu.async_copy(a, b, sem_e)

    prev = None
    for idx in range(NBLK):
        g, h = divmod(idx, H)
        wsh = wsh0 if g == 0 else wsh1
        dinv_t = di_a if idx % 2 == 0 else di_b
        vbuf, ibuf = (vbufa, ibufa) if idx % 2 == 0 else (vbufb, ibufb)
        if (g, h) == (1, 0):
            pltpu.sync_copy(src2.at[pl.ds(w * EPT, EPT)],
                            src_t.at[pl.ds(0, EPT)])
            pltpu.sync_copy(dst2.at[pl.ds(w * EPT, EPT)],
                            dst_t.at[pl.ds(0, EPT)])
        a, b = dtbl(idx)
        pltpu.make_async_copy(a, b, sem_t).wait()
        if idx + 1 < NBLK:
            a, b = dtbl(idx + 1)
            pltpu.async_copy(a, b, sem_t)
        a, b = ebl(idx)
        pltpu.make_async_copy(a, b, sem_e).wait()
        hoff = h * STR

        @plsc.parallel_loop(0, NIT2, unroll=16)
        def _edge(i):
            off = i * 16
            s16 = src_t[pl.ds(off, 16)]
            d16 = dst_t[pl.ds(off, 16)]
            dv = plsc.load_gather(dinv_t, [d16])
            vbuf[i // VPR, pl.ds((i % VPR) * 16, 16)] = (
                vbuf[i // VPR, pl.ds((i % VPR) * 16, 16)] * dv)
            ibuf[i // VPR, pl.ds((i % VPR) * 16, 16)] = s16 + hoff

        if prev is not None:
            pv, pi, pd = prev

            def _drain(r, _):
                pltpu.make_async_copy(pv.at[r], pd.at[pi.at[r]],
                                      sem_d).wait()
                return 0
            lax.fori_loop(0, NCH, _drain, 0)
        if idx + 1 < NBLK:
            a, b = ebl(idx + 1)
            pltpu.async_copy(a, b, sem_e)

        def _fire(r, _):
            pltpu.async_copy(vbuf.at[r], wsh.at[ibuf.at[r]], sem_d,
                             add=True)
            return 0
        lax.fori_loop(0, NCH, _fire, 0)
        prev = (vbuf, ibuf, wsh)

    pv, pi, pd = prev

    def _drain_last(r, _):
        pltpu.make_async_copy(pv.at[r], pd.at[pi.at[r]], sem_d).wait()
        return 0
    lax.fori_loop(0, NCH, _drain_last, 0)
    plsc.subcore_barrier()

    for g in range(2):
        wsh = wsh0 if g == 0 else wsh1
        out_c0 = w1c0 if g == 0 else w2c0
        out_c1 = w1c1 if g == 0 else w2c1
        pltpu.sync_copy(wsh.at[pl.ds(s * SPT, SPT)], stg0)

        @pl.when(c == 0)
        def _():
            pltpu.sync_copy(stg0, out_c0.at[pl.ds(s * SPT, SPT)])

        @pl.when(c == 1)
        def _():
            pltpu.sync_copy(stg0, out_c1.at[pl.ds(s * SPT, SPT)])


def _tc_epilogue(x1, x2, W1, W2, Wl, b1f, b2f, blf,
                 w1a, w1b, w2a, w2b, o1, o2):
    rowh = lax.broadcasted_iota(jnp.int32, (H, H * D), 0)
    colh = lax.broadcasted_iota(jnp.int32, (H, H * D), 1) // D
    means = []
    for x, W, bf, wa, wb in ((x1, W1, b1f, w1a, w1b),
                             (x2, W2, b2f, w2a, w2b)):
        w2d = wa[...] + wb[...]
        u = lax.dot_general(w2d, x[...], (((1,), (0,)), ((), ())),
                            preferred_element_type=jnp.float32)
        P = jnp.dot(u, W[...], preferred_element_type=jnp.float32)
        msel = jnp.where(rowh == colh, P, 0.0)
        mean_flat = jnp.sum(msel, axis=0, keepdims=True) / NN + bf[...]
        means.append(mean_flat)
    o1[...] = jnp.dot(means[1], Wl[...],
                      preferred_element_type=jnp.float32) + blf[...]
    o2[...] = jnp.dot(means[0], Wl[...],
                      preferred_element_type=jnp.float32) + blf[...]


def _run_tc_epilogue(x1, x2, W1, W2, Wl, b1f, b2f, blf, w1a, w1b, w2a, w2b):
    shp = jax.ShapeDtypeStruct((1, 128), jnp.float32)
    return pl.pallas_call(
        _tc_epilogue,
        out_shape=[shp, shp],
    )(x1, x2, W1, W2, Wl, b1f, b2f, blf, w1a, w1b, w2a, w2b)


def kernel(x1, x2, edge_index1, edge_index2, W1, a_src1, a_dst1, b1,
           W2, a_src2, a_dst2, b2, Wl, bl):
    x1 = x1.astype(jnp.float32)
    x2 = x2.astype(jnp.float32)
    pad = jnp.full((EPAD - EE,), NN, jnp.int32)
    src1 = jnp.concatenate([edge_index1[0].astype(jnp.int32), pad])
    dst1 = jnp.concatenate([edge_index1[1].astype(jnp.int32), pad])
    src2 = jnp.concatenate([edge_index2[0].astype(jnp.int32), pad])
    dst2 = jnp.concatenate([edge_index2[1].astype(jnp.int32), pad])

    als, ald, ini = _run_tc_prologue(
        x1, x2, W1, a_src1.reshape(1, H * D), a_dst1.reshape(1, H * D),
        W2, a_src2.reshape(1, H * D), a_dst2.reshape(1, H * D))
    als1 = als[0].reshape(H * NN)
    ald1 = ald[0].reshape(H * NN)
    als2 = als[1].reshape(H * NN)
    ald2 = ald[1].reshape(H * NN)
    inip = jnp.pad(ini, ((0, 0), (0, 0), (0, STR - NN))).reshape(2, FPAD)
    ini1 = inip[0]
    ini2 = inip[1]

    d1c0, d1c1, d2c0, d2c1, eeb = _sc_pass1(
        src1, dst1, src2, dst2, als1, ald1, als2, ald2, ini1, ini2)
    w1c0, w1c1, w2c0, w2c1 = _sc_pass2(
        src1, dst1, src2, dst2, ini1, ini2,
        d1c0, d1c1, d2c0, d2c1, eeb)

    def _w2d(v):
        return v.reshape(H, STR)[:, :NN]

    o1, o2 = _run_tc_epilogue(
        x1, x2, W1, W2, Wl,
        b1.reshape(1, H * D), b2.reshape(1, H * D), bl.reshape(1, 128),
        _w2d(w1c0), _w2d(w1c1), _w2d(w2c0), _w2d(w2c1))
    return (o1.reshape(128), o2.reshape(128))


# speedup vs baseline: 1.0232x; 1.0232x over previous
import functools

import jax
import jax.numpy as jnp
from jax import lax
from jax.experimental import pallas as pl
from jax.experimental.pallas import tpu as pltpu
from jax.experimental.pallas import tpu_sc as plsc

H = 8
D = 128
IND = 128
NN = 10000
EE = 160000

NC = 2
NS = 16
NW = NC * NS
EPT = 5008
EPAD = NW * EPT
STR = NN + 16
FPAD = H * STR
SPT = FPAD // NS
NCH = 40
CW = 128
VPR = CW // 16
TLE = NCH * CW
NIT2 = NCH * VPR
NBLK = 2 * H


def _tc_prologue(x1, x2, W1, as1, ad1, W2, as2, ad2, als_o, ald_o, ini_o):
    hd_iota = lax.broadcasted_iota(jnp.int32, (H * D, H), 0) // D
    h_iota = lax.broadcasted_iota(jnp.int32, (H * D, H), 1)
    B = jnp.where(hd_iota == h_iota, 1.0, 0.0)
    for g, (x, W, asv, adv) in enumerate(((x1, W1, as1, ad1),
                                          (x2, W2, as2, ad2))):
        xv = x[...]
        Wv = W[...]
        A_s = jnp.dot(Wv * asv[...], B, preferred_element_type=jnp.float32)
        A_d = jnp.dot(Wv * adv[...], B, preferred_element_type=jnp.float32)
        als = lax.dot_general(A_s, xv, (((0,), (1,)), ((), ())),
                              preferred_element_type=jnp.float32)
        ald = lax.dot_general(A_d, xv, (((0,), (1,)), ((), ())),
                              preferred_element_type=jnp.float32)
        z = als + ald
        ini_o[g] = jnp.exp(jnp.maximum(z, 0.2 * z))
        als_o[g] = als
        ald_o[g] = ald


def _run_tc_prologue(x1, x2, W1, as1f, ad1f, W2, as2f, ad2f):
    shp = jax.ShapeDtypeStruct((2, H, NN), jnp.float32)
    return pl.pallas_call(
        _tc_prologue,
        out_shape=[shp, shp, shp],
    )(x1, x2, W1, as1f, ad1f, W2, as2f, ad2f)


_SC_MESH = plsc.VectorSubcoreMesh(core_axis_name="c", subcore_axis_name="s")

_F1 = jax.ShapeDtypeStruct((FPAD,), jnp.float32)
_R1 = jax.ShapeDtypeStruct((H * NN,), jnp.float32)
_EB = jax.ShapeDtypeStruct((2 * H * NW, NCH, CW), jnp.float32)


def _edge_tail_init(src_t, dst_t):
    pad16 = jnp.full((16,), NN, jnp.int32)
    for k in range(EPT, TLE, 16):
        src_t[pl.ds(k, 16)] = pad16
        dst_t[pl.ds(k, 16)] = pad16


@functools.partial(
    pl.kernel,
    out_type=[_F1, _F1, _F1, _F1, _EB],
    mesh=_SC_MESH,
    compiler_params=pltpu.CompilerParams(needs_layout_passes=False),
    scratch_types=[
        pltpu.VMEM((STR,), jnp.float32),
        pltpu.VMEM((STR,), jnp.float32),
        pltpu.VMEM((STR,), jnp.float32),
        pltpu.VMEM((STR,), jnp.float32),
        pltpu.VMEM((TLE,), jnp.int32),
        pltpu.VMEM((TLE,), jnp.int32),
        pltpu.VMEM((NCH, CW), jnp.float32),
        pltpu.VMEM((NCH, CW), jnp.int32),
        pltpu.VMEM((NCH, CW), jnp.float32),
        pltpu.VMEM((NCH, CW), jnp.int32),
        pltpu.VMEM((SPT,), jnp.float32),
        pltpu.VMEM_SHARED((FPAD,), jnp.float32),
        pltpu.VMEM_SHARED((FPAD,), jnp.float32),
        pltpu.SemaphoreType.DMA,
        pltpu.SemaphoreType.DMA,
        pltpu.SemaphoreType.DMA,
    ],
)
def _sc_pass1(src1, dst1, src2, dst2, als1, ald1, als2, ald2, ini1, ini2,
              d1c0, d1c1, d2c0, d2c1, eeb,
              as_a, ad_a, as_b, ad_b, src_t, dst_t,
              vbufa, ibufa, vbufb, ibufb,
              stg, dsh0, dsh1, sem_d, sem_t, sem_e):
    c = lax.axis_index("c")
    s = lax.axis_index("s")
    w = c * NS + s

    _edge_tail_init(src_t, dst_t)

    factor = jnp.where(c == 0, 1.0, 0.0).astype(jnp.float32)
    for g in range(2):
        dsh = dsh0 if g == 0 else dsh1
        ini = ini1 if g == 0 else ini2
        pltpu.sync_copy(ini.at[pl.ds(s * SPT, SPT)], stg)

        @plsc.parallel_loop(0, SPT // 16, unroll=8)
        def _scale(i):
            stg[pl.ds(i * 16, 16)] = stg[pl.ds(i * 16, 16)] * factor
        pltpu.sync_copy(stg, dsh.at[pl.ds(s * SPT, SPT)])
    plsc.subcore_barrier()

    def tbl(idx):
        g, h = divmod(idx, H)
        a = (als1 if g == 0 else als2).at[pl.ds(h * NN, NN)]
        b = (ald1 if g == 0 else ald2).at[pl.ds(h * NN, NN)]
        t = (as_a, ad_a) if idx % 2 == 0 else (as_b, ad_b)
        return (a, t[0].at[pl.ds(0, NN)]), (b, t[1].at[pl.ds(0, NN)])

    pltpu.sync_copy(src1.at[pl.ds(w * EPT, EPT)], src_t.at[pl.ds(0, EPT)])
    pltpu.sync_copy(dst1.at[pl.ds(w * EPT, EPT)], dst_t.at[pl.ds(0, EPT)])
    for pair in tbl(0):
        pltpu.async_copy(pair[0], pair[1], sem_t)

    prev = None
    for idx in range(NBLK):
        g, h = divmod(idx, H)
        dsh = dsh0 if g == 0 else dsh1
        as_t, ad_t = (as_a, ad_a) if idx % 2 == 0 else (as_b, ad_b)
        vbuf, ibuf = (vbufa, ibufa) if idx % 2 == 0 else (vbufb, ibufb)
        if (g, h) == (1, 0):
            pltpu.sync_copy(src2.at[pl.ds(w * EPT, EPT)],
                            src_t.at[pl.ds(0, EPT)])
            pltpu.sync_copy(dst2.at[pl.ds(w * EPT, EPT)],
                            dst_t.at[pl.ds(0, EPT)])
        for pair in tbl(idx):
            pltpu.make_async_copy(pair[0], pair[1], sem_t).wait()
        if idx + 1 < NBLK:
            for pair in tbl(idx + 1):
                pltpu.async_copy(pair[0], pair[1], sem_t)
        hoff = h * STR

        @plsc.parallel_loop(0, NIT2, unroll=8)
        def _edge(i):
            off = i * 16
            s16 = src_t[pl.ds(off, 16)]
            d16 = dst_t[pl.ds(off, 16)]
            z = (plsc.load_gather(as_t, [s16])
                 + plsc.load_gather(ad_t, [d16]))
            ee = jnp.exp(jnp.maximum(z, 0.2 * z))
            vbuf[i // VPR, pl.ds((i % VPR) * 16, 16)] = ee
            ibuf[i // VPR, pl.ds((i % VPR) * 16, 16)] = d16 + hoff

        if prev is not None:
            pv, pi, pd, pblk = prev

            def _drain(r, _):
                pltpu.make_async_copy(pv.at[r], pd.at[pi.at[r]],
                                      sem_d).wait()
                return 0
            lax.fori_loop(0, NCH, _drain, 0)
            pltpu.make_async_copy(pv, eeb.at[pblk], sem_e).wait()

        def _fire(r, _):
            pltpu.async_copy(vbuf.at[r], dsh.at[ibuf.at[r]], sem_d,
                             add=True)
            return 0
        lax.fori_loop(0, NCH, _fire, 0)
        blk = idx * NW + w
        pltpu.async_copy(vbuf, eeb.at[blk], sem_e)
        prev = (vbuf, ibuf, dsh, blk)

    pv, pi, pd, pblk = prev

    def _drain_last(r, _):
        pltpu.make_async_copy(pv.at[r], pd.at[pi.at[r]], sem_d).wait()
        return 0
    lax.fori_loop(0, NCH, _drain_last, 0)
    pltpu.make_async_copy(pv, eeb.at[pblk], sem_e).wait()
    plsc.subcore_barrier()

    for g in range(2):
        dsh = dsh0 if g == 0 else dsh1
        out_c0 = d1c0 if g == 0 else d2c0
        out_c1 = d1c1 if g == 0 else d2c1
        pltpu.sync_copy(dsh.at[pl.ds(s * SPT, SPT)], stg)

        @pl.when(c == 0)
        def _():
            pltpu.sync_copy(stg, out_c0.at[pl.ds(s * SPT, SPT)])

        @pl.when(c == 1)
        def _():
            pltpu.sync_copy(stg, out_c1.at[pl.ds(s * SPT, SPT)])


@functools.partial(
    pl.kernel,
    out_type=[_R1, _R1, _R1, _R1],
    mesh=_SC_MESH,
    compiler_params=pltpu.CompilerParams(needs_layout_passes=False),
    scratch_types=[
        pltpu.VMEM((STR,), jnp.float32),
        pltpu.VMEM((STR,), jnp.float32),
        pltpu.VMEM((TLE,), jnp.int32),
        pltpu.VMEM((TLE,), jnp.int32),
        pltpu.VMEM((NCH, CW), jnp.float32),
        pltpu.VMEM((NCH, CW), jnp.int32),
        pltpu.VMEM((NCH, CW), jnp.float32),
        pltpu.VMEM((NCH, CW), jnp.int32),
        pltpu.VMEM((SPT,), jnp.float32),
        pltpu.VMEM((SPT,), jnp.float32),
        pltpu.VMEM((SPT,), jnp.float32),
        pltpu.VMEM_SHARED((FPAD,), jnp.float32),
        pltpu.VMEM_SHARED((FPAD,), jnp.float32),
        pltpu.VMEM_SHARED((FPAD,), jnp.float32),
        pltpu.VMEM_SHARED((FPAD,), jnp.float32),
        pltpu.SemaphoreType.DMA,
        pltpu.SemaphoreType.DMA,
        pltpu.SemaphoreType.DMA,
    ],
)
def _sc_pass2(src1, dst1, src2, dst2, ini1, ini2,
              d1c0, d1c1, d2c0, d2c1, eeb,
              w1c0, w1c1, w2c0, w2c1,
              di_a, di_b, src_t, dst_t,
              vbufa, ibufa, vbufb, ibufb,
              stg0, stg1, stg2, ish0, ish1, wsh0, wsh1,
              sem_d, sem_t, sem_e):
    c = lax.axis_index("c")
    s = lax.axis_index("s")
    w = c * NS + s

    _edge_tail_init(src_t, dst_t)

    factor = jnp.where(c == 0, 1.0, 0.0).astype(jnp.float32)
    for g in range(2):
        ish = ish0 if g == 0 else ish1
        wsh = wsh0 if g == 0 else wsh1
        dp0 = d1c0 if g == 0 else d2c0
        dp1 = d1c1 if g == 0 else d2c1
        ini = ini1 if g == 0 else ini2
        pltpu.sync_copy(dp0.at[pl.ds(s * SPT, SPT)], stg0)
        pltpu.sync_copy(dp1.at[pl.ds(s * SPT, SPT)], stg1)
        pltpu.sync_copy(ini.at[pl.ds(s * SPT, SPT)], stg2)

        @plsc.parallel_loop(0, SPT // 16, unroll=8)
        def _inv(i):
            dv = 1.0 / (stg0[pl.ds(i * 16, 16)] + stg1[pl.ds(i * 16, 16)])
            stg0[pl.ds(i * 16, 16)] = dv
            stg1[pl.ds(i * 16, 16)] = stg2[pl.ds(i * 16, 16)] * dv * factor
        pltpu.sync_copy(stg0, ish.at[pl.ds(s * SPT, SPT)])
        pltpu.sync_copy(stg1, wsh.at[pl.ds(s * SPT, SPT)])
    plsc.subcore_barrier()

    def dtbl(idx):
        g, h = divmod(idx, H)
        ish = ish0 if g == 0 else ish1
        t = di_a if idx % 2 == 0 else di_b
        return ish.at[pl.ds(h * STR, STR)], t

    def ebl(idx):
        vbuf = vbufa if idx % 2 == 0 else vbufb
        return eeb.at[idx * NW + w], vbuf

    pltpu.sync_copy(src1.at[pl.ds(w * EPT, EPT)], src_t.at[pl.ds(0, EPT)])
    pltpu.sync_copy(dst1.at[pl.ds(w * EPT, EPT)], dst_t.at[pl.ds(0, EPT)])
    a, b = dtbl(0)
    pltpu.async_copy(a, b, sem_t)
    a, b = ebl(0)
    pltpu.async_copy(a, b, sem_e)

    prev = None
    for idx in range(NBLK):
        g, h = divmod(idx, H)
        wsh = wsh0 if g == 0 else wsh1
        dinv_t = di_a if idx % 2 == 0 else di_b
        vbuf, ibuf = (vbufa, ibufa) if idx % 2 == 0 else (vbufb, ibufb)
        if (g, h) == (1, 0):
            pltpu.sync_copy(src2.at[pl.ds(w * EPT, EPT)],
                            src_t.at[pl.ds(0, EPT)])
            pltpu.sync_copy(dst2.at[pl.ds(w * EPT, EPT)],
                            dst_t.at[pl.ds(0, EPT)])
        a, b = dtbl(idx)
        pltpu.make_async_copy(a, b, sem_t).wait()
        if idx + 1 < NBLK:
            a, b = dtbl(idx + 1)
            pltpu.async_copy(a, b, sem_t)
        a, b = ebl(idx)
        pltpu.make_async_copy(a, b, sem_e).wait()
        hoff = h * STR

        @plsc.parallel_loop(0, NIT2, unroll=8)
        def _edge(i):
            off = i * 16
            s16 = src_t[pl.ds(off, 16)]
            d16 = dst_t[pl.ds(off, 16)]
            dv = plsc.load_gather(dinv_t, [d16])
            vbuf[i // VPR, pl.ds((i % VPR) * 16, 16)] = (
                vbuf[i // VPR, pl.ds((i % VPR) * 16, 16)] * dv)
            ibuf[i // VPR, pl.ds((i % VPR) * 16, 16)] = s16 + hoff

        if prev is not None:
            pv, pi, pd = prev

            def _drain(r, _):
                pltpu.make_async_copy(pv.at[r], pd.at[pi.at[r]],
                                      sem_d).wait()
                return 0
            lax.fori_loop(0, NCH, _drain, 0)
        if idx + 1 < NBLK:
            a, b = ebl(idx + 1)
            pltpu.async_copy(a, b, sem_e)

        def _fire(r, _):
            pltpu.async_copy(vbuf.at[r], wsh.at[ibuf.at[r]], sem_d,
                             add=True)
            return 0
        lax.fori_loop(0, NCH, _fire, 0)
        prev = (vbuf, ibuf, wsh)

    pv, pi, pd = prev

    def _drain_last(r, _):
        pltpu.make_async_copy(pv.at[r], pd.at[pi.at[r]], sem_d).wait()
        return 0
    lax.fori_loop(0, NCH, _drain_last, 0)
    plsc.subcore_barrier()

    soff = (s // 2) * STR + (s % 2) * (NN // 2)
    doff = s * (NN // 2)
    for g in range(2):
        wsh = wsh0 if g == 0 else wsh1
        out_c0 = w1c0 if g == 0 else w2c0
        out_c1 = w1c1 if g == 0 else w2c1
        pltpu.sync_copy(wsh.at[pl.ds(soff, NN // 2)], stg0.at[pl.ds(0, NN // 2)])

        @pl.when(c == 0)
        def _():
            pltpu.sync_copy(stg0.at[pl.ds(0, NN // 2)], out_c0.at[pl.ds(doff, NN // 2)])

        @pl.when(c == 1)
        def _():
            pltpu.sync_copy(stg0.at[pl.ds(0, NN // 2)], out_c1.at[pl.ds(doff, NN // 2)])


def _tc_epilogue(x1, x2, W1, W2, Wl, b1f, b2f, blf,
                 w1a, w1b, w2a, w2b, o1, o2):
    rowh = lax.broadcasted_iota(jnp.int32, (H, H * D), 0)
    colh = lax.broadcasted_iota(jnp.int32, (H, H * D), 1) // D
    means = []
    for x, W, bf, wa, wb in ((x1, W1, b1f, w1a, w1b),
                             (x2, W2, b2f, w2a, w2b)):
        w2d = wa[...] + wb[...]
        u = lax.dot_general(w2d, x[...], (((1,), (0,)), ((), ())),
                            preferred_element_type=jnp.float32)
        P = jnp.dot(u, W[...], preferred_element_type=jnp.float32)
        msel = jnp.where(rowh == colh, P, 0.0)
        mean_flat = jnp.sum(msel, axis=0, keepdims=True) / NN + bf[...]
        means.append(mean_flat)
    o1[...] = jnp.dot(means[1], Wl[...],
                      preferred_element_type=jnp.float32) + blf[...]
    o2[...] = jnp.dot(means[0], Wl[...],
                      preferred_element_type=jnp.float32) + blf[...]


def _run_tc_epilogue(x1, x2, W1, W2, Wl, b1f, b2f, blf, w1a, w1b, w2a, w2b):
    shp = jax.ShapeDtypeStruct((1, 128), jnp.float32)
    return pl.pallas_call(
        _tc_epilogue,
        out_shape=[shp, shp],
    )(x1, x2, W1, W2, Wl, b1f, b2f, blf, w1a, w1b, w2a, w2b)


def kernel(x1, x2, edge_index1, edge_index2, W1, a_src1, a_dst1, b1,
           W2, a_src2, a_dst2, b2, Wl, bl):
    x1 = x1.astype(jnp.float32)
    x2 = x2.astype(jnp.float32)
    pad = jnp.full((EPAD - EE,), NN, jnp.int32)
    src1 = jnp.concatenate([edge_index1[0].astype(jnp.int32), pad])
    dst1 = jnp.concatenate([edge_index1[1].astype(jnp.int32), pad])
    src2 = jnp.concatenate([edge_index2[0].astype(jnp.int32), pad])
    dst2 = jnp.concatenate([edge_index2[1].astype(jnp.int32), pad])

    als, ald, ini = _run_tc_prologue(
        x1, x2, W1, a_src1.reshape(1, H * D), a_dst1.reshape(1, H * D),
        W2, a_src2.reshape(1, H * D), a_dst2.reshape(1, H * D))
    als1 = als[0].reshape(H * NN)
    ald1 = ald[0].reshape(H * NN)
    als2 = als[1].reshape(H * NN)
    ald2 = ald[1].reshape(H * NN)
    inip = jnp.pad(ini, ((0, 0), (0, 0), (0, STR - NN))).reshape(2, FPAD)
    ini1 = inip[0]
    ini2 = inip[1]

    d1c0, d1c1, d2c0, d2c1, eeb = _sc_pass1(
        src1, dst1, src2, dst2, als1, ald1, als2, ald2, ini1, ini2)
    w1c0, w1c1, w2c0, w2c1 = _sc_pass2(
        src1, dst1, src2, dst2, ini1, ini2,
        d1c0, d1c1, d2c0, d2c1, eeb)

    def _w2d(v):
        return v.reshape(H, NN)

    o1, o2 = _run_tc_epilogue(
        x1, x2, W1, W2, Wl,
        b1.reshape(1, H * D), b2.reshape(1, H * D), bl.reshape(1, 128),
        _w2d(w1c0), _w2d(w1c1), _w2d(w2c0), _w2d(w2c1))
    return (o1.reshape(128), o2.reshape(128))


# speedup vs baseline: 1.3003x; 1.2708x over previous
import functools

import jax
import jax.numpy as jnp
from jax import lax
from jax.experimental import pallas as pl
from jax.experimental.pallas import tpu as pltpu
from jax.experimental.pallas import tpu_sc as plsc

H = 8
D = 128
IND = 128
NN = 10000
EE = 160000

NC = 2
NS = 16
HPC = H // NC
EPT = EE // NS
STR = NN + 16
FLAT = H * NN
FPAD = H * STR
CSTR = HPC * STR
CSPT = CSTR // NS
NCH = 79
CW = 128
TLE = NCH * CW
NIT = NCH * 8
NBLK = 2 * HPC


def _tc_prologue(x1, x2, W1, as1, ad1, W2, as2, ad2, als_o, ald_o, ini_o):
    hd_iota = lax.broadcasted_iota(jnp.int32, (H * D, H), 0) // D
    h_iota = lax.broadcasted_iota(jnp.int32, (H * D, H), 1)
    B = jnp.where(hd_iota == h_iota, 1.0, 0.0)
    for g, (x, W, asv, adv) in enumerate(((x1, W1, as1, ad1),
                                          (x2, W2, as2, ad2))):
        xv = x[...]
        Wv = W[...]
        A_s = jnp.dot(Wv * asv[...], B, preferred_element_type=jnp.float32)
        A_d = jnp.dot(Wv * adv[...], B, preferred_element_type=jnp.float32)
        als = lax.dot_general(A_s, xv, (((0,), (1,)), ((), ())),
                              preferred_element_type=jnp.float32)
        ald = lax.dot_general(A_d, xv, (((0,), (1,)), ((), ())),
                              preferred_element_type=jnp.float32)
        z = als + ald
        ini_o[g] = jnp.exp(jnp.maximum(z, 0.2 * z))
        als_o[g] = als
        ald_o[g] = ald


def _run_tc_prologue(x1, x2, W1, as1f, ad1f, W2, as2f, ad2f):
    shp = jax.ShapeDtypeStruct((2, H, NN), jnp.float32)
    return pl.pallas_call(
        _tc_prologue,
        out_shape=[shp, shp, shp],
    )(x1, x2, W1, as1f, ad1f, W2, as2f, ad2f)


_SC_MESH = plsc.VectorSubcoreMesh(core_axis_name="c", subcore_axis_name="s")

_R1 = jax.ShapeDtypeStruct((FLAT,), jnp.float32)


@functools.partial(
    pl.kernel,
    out_type=[_R1, _R1],
    mesh=_SC_MESH,
    compiler_params=pltpu.CompilerParams(needs_layout_passes=False),
    scratch_types=[
        pltpu.VMEM((STR,), jnp.float32),
        pltpu.VMEM((STR,), jnp.float32),
        pltpu.VMEM((STR,), jnp.float32),
        pltpu.VMEM((STR,), jnp.float32),
        pltpu.VMEM((TLE,), jnp.int32),
        pltpu.VMEM((TLE,), jnp.int32),
        pltpu.VMEM((NCH, CW), jnp.float32),
        pltpu.VMEM((NCH, CW), jnp.int32),
        pltpu.VMEM((NCH, CW), jnp.float32),
        pltpu.VMEM((NCH, CW), jnp.int32),
        pltpu.VMEM((5008,), jnp.float32),
        pltpu.VMEM((CSPT,), jnp.float32),
        pltpu.VMEM_SHARED((CSTR,), jnp.float32),
        pltpu.VMEM_SHARED((CSTR,), jnp.float32),
        pltpu.VMEM_SHARED((CSTR,), jnp.float32),
        pltpu.VMEM_SHARED((CSTR,), jnp.float32),
        pltpu.VMEM_SHARED((CSTR,), jnp.float32),
        pltpu.VMEM_SHARED((CSTR,), jnp.float32),
        pltpu.HBM((NC * NBLK * NS, NCH, CW), jnp.float32),
        pltpu.SemaphoreType.DMA,
        pltpu.SemaphoreType.DMA,
        pltpu.SemaphoreType.DMA,
    ],
)
def _sc_gat(src1, dst1, src2, dst2, als1, ald1, als2, ald2, ini1, ini2,
            w1, w2,
            as_a, ad_a, as_b, ad_b, src_t, dst_t,
            vbufa, ibufa, vbufb, ibufb, stg0, stg1,
            dsh0, dsh1, ish0, ish1, wsh0, wsh1, eesh,
            sem_d, sem_t, sem_e):
    c = lax.axis_index("c")
    s = lax.axis_index("s")

    pad16 = jnp.full((16,), NN, jnp.int32)
    for k in range(EPT, TLE, 16):
        src_t[pl.ds(k, 16)] = pad16
        dst_t[pl.ds(k, 16)] = pad16

    for g in range(2):
        dsh = dsh0 if g == 0 else dsh1
        ini = ini1 if g == 0 else ini2
        ioff = pl.multiple_of(c * CSTR + s * CSPT, 8)
        pltpu.sync_copy(ini.at[pl.ds(ioff, CSPT)], stg1)
        pltpu.sync_copy(stg1, dsh.at[pl.ds(s * CSPT, CSPT)])
    plsc.subcore_barrier()

    def tbl(idx):
        g, j = divmod(idx, HPC)
        toff = pl.multiple_of((c * HPC + j) * NN, 8)
        a = (als1 if g == 0 else als2).at[pl.ds(toff, NN)]
        b = (ald1 if g == 0 else ald2).at[pl.ds(toff, NN)]
        t = (as_a, ad_a) if idx % 2 == 0 else (as_b, ad_b)
        return (a, t[0].at[pl.ds(0, NN)]), (b, t[1].at[pl.ds(0, NN)])

    pltpu.sync_copy(src1.at[pl.ds(s * EPT, EPT)], src_t.at[pl.ds(0, EPT)])
    pltpu.sync_copy(dst1.at[pl.ds(s * EPT, EPT)], dst_t.at[pl.ds(0, EPT)])
    for pair in tbl(0):
        pltpu.async_copy(pair[0], pair[1], sem_t)

    prev = None
    for idx in range(NBLK):
        g, j = divmod(idx, HPC)
        dsh = dsh0 if g == 0 else dsh1
        as_t, ad_t = (as_a, ad_a) if idx % 2 == 0 else (as_b, ad_b)
        vbuf, ibuf = (vbufa, ibufa) if idx % 2 == 0 else (vbufb, ibufb)
        if (g, j) == (1, 0):
            pltpu.sync_copy(src2.at[pl.ds(s * EPT, EPT)],
                            src_t.at[pl.ds(0, EPT)])
            pltpu.sync_copy(dst2.at[pl.ds(s * EPT, EPT)],
                            dst_t.at[pl.ds(0, EPT)])
        for pair in tbl(idx):
            pltpu.make_async_copy(pair[0], pair[1], sem_t).wait()
        if idx + 1 < NBLK:
            for pair in tbl(idx + 1):
                pltpu.async_copy(pair[0], pair[1], sem_t)
        hoff = j * STR

        @plsc.parallel_loop(0, NIT, unroll=8)
        def _edge(i):
            off = i * 16
            s16 = src_t[pl.ds(off, 16)]
            d16 = dst_t[pl.ds(off, 16)]
            z = (plsc.load_gather(as_t, [s16])
                 + plsc.load_gather(ad_t, [d16]))
            ee = jnp.exp(jnp.maximum(z, 0.2 * z))
            vbuf[i // 8, pl.ds((i % 8) * 16, 16)] = ee
            ibuf[i // 8, pl.ds((i % 8) * 16, 16)] = d16 + hoff

        if prev is not None:
            pv, pi, pd, pblk = prev

            def _drain(r, _):
                pltpu.make_async_copy(pv.at[r], pd.at[pi.at[r]],
                                      sem_d).wait()
                return 0
            lax.fori_loop(0, NCH, _drain, 0)
            pltpu.make_async_copy(pv, eesh.at[pblk], sem_e).wait()

        def _fire(r, _):
            pltpu.async_copy(vbuf.at[r], dsh.at[ibuf.at[r]], sem_d,
                             add=True)
            return 0
        lax.fori_loop(0, NCH, _fire, 0)
        blk = (c * NBLK + idx) * NS + s
        pltpu.async_copy(vbuf, eesh.at[blk], sem_e)
        prev = (vbuf, ibuf, dsh, blk)

    pv, pi, pd, pblk = prev

    def _drain_last(r, _):
        pltpu.make_async_copy(pv.at[r], pd.at[pi.at[r]], sem_d).wait()
        return 0
    lax.fori_loop(0, NCH, _drain_last, 0)
    pltpu.make_async_copy(pv, eesh.at[pblk], sem_e).wait()
    plsc.subcore_barrier()

    for g in range(2):
        dsh = dsh0 if g == 0 else dsh1
        ish = ish0 if g == 0 else ish1
        wsh = wsh0 if g == 0 else wsh1
        ini = ini1 if g == 0 else ini2
        pltpu.sync_copy(dsh.at[pl.ds(s * CSPT, CSPT)],
                        stg0.at[pl.ds(0, CSPT)])
        ioff = pl.multiple_of(c * CSTR + s * CSPT, 8)
        pltpu.sync_copy(ini.at[pl.ds(ioff, CSPT)], stg1)

        @plsc.parallel_loop(0, CSPT // 16, unroll=8)
        def _inv(i):
            dv = 1.0 / stg0[pl.ds(i * 16, 16)]
            stg0[pl.ds(i * 16, 16)] = dv
            stg1[pl.ds(i * 16, 16)] = stg1[pl.ds(i * 16, 16)] * dv
        pltpu.sync_copy(stg0.at[pl.ds(0, CSPT)],
                        ish.at[pl.ds(s * CSPT, CSPT)])
        pltpu.sync_copy(stg1, wsh.at[pl.ds(s * CSPT, CSPT)])
    plsc.subcore_barrier()

    def dtbl(idx):
        g, j = divmod(idx, HPC)
        ish = ish0 if g == 0 else ish1
        t = as_a if idx % 2 == 0 else as_b
        return ish.at[pl.ds(j * STR, STR)], t

    def ebl(idx):
        vbuf = vbufa if idx % 2 == 0 else vbufb
        return eesh.at[(c * NBLK + idx) * NS + s], vbuf

    pltpu.sync_copy(src1.at[pl.ds(s * EPT, EPT)], src_t.at[pl.ds(0, EPT)])
    pltpu.sync_copy(dst1.at[pl.ds(s * EPT, EPT)], dst_t.at[pl.ds(0, EPT)])
    a, b = dtbl(0)
    pltpu.async_copy(a, b, sem_t)
    a, b = ebl(0)
    pltpu.async_copy(a, b, sem_e)

    prev = None
    for idx in range(NBLK):
        g, j = divmod(idx, HPC)
        wsh = wsh0 if g == 0 else wsh1
        dinv_t = as_a if idx % 2 == 0 else as_b
        vbuf, ibuf = (vbufa, ibufa) if idx % 2 == 0 else (vbufb, ibufb)
        if (g, j) == (1, 0):
            pltpu.sync_copy(src2.at[pl.ds(s * EPT, EPT)],
                            src_t.at[pl.ds(0, EPT)])
            pltpu.sync_copy(dst2.at[pl.ds(s * EPT, EPT)],
                            dst_t.at[pl.ds(0, EPT)])
        a, b = dtbl(idx)
        pltpu.make_async_copy(a, b, sem_t).wait()
        if idx + 1 < NBLK:
            a, b = dtbl(idx + 1)
            pltpu.async_copy(a, b, sem_t)
        a, b = ebl(idx)
        pltpu.make_async_copy(a, b, sem_e).wait()
        hoff = j * STR

        @plsc.parallel_loop(0, NIT, unroll=8)
        def _edge(i):
            off = i * 16
            s16 = src_t[pl.ds(off, 16)]
            d16 = dst_t[pl.ds(off, 16)]
            dv = plsc.load_gather(dinv_t, [d16])
            vbuf[i // 8, pl.ds((i % 8) * 16, 16)] = (
                vbuf[i // 8, pl.ds((i % 8) * 16, 16)] * dv)
            ibuf[i // 8, pl.ds((i % 8) * 16, 16)] = s16 + hoff

        if prev is not None:
            pv, pi, pd = prev

            def _drain(r, _):
                pltpu.make_async_copy(pv.at[r], pd.at[pi.at[r]],
                                      sem_d).wait()
                return 0
            lax.fori_loop(0, NCH, _drain, 0)
        if idx + 1 < NBLK:
            a, b = ebl(idx + 1)
            pltpu.async_copy(a, b, sem_e)

        def _fire(r, _):
            pltpu.async_copy(vbuf.at[r], wsh.at[ibuf.at[r]], sem_d,
                             add=True)
            return 0
        lax.fori_loop(0, NCH, _fire, 0)
        prev = (vbuf, ibuf, wsh)

    pv, pi, pd = prev

    def _drain_last2(r, _):
        pltpu.make_async_copy(pv.at[r], pd.at[pi.at[r]], sem_d).wait()
        return 0
    lax.fori_loop(0, NCH, _drain_last2, 0)
    plsc.subcore_barrier()

    @pl.when(s < 8)
    def _():
        j = s // 2
        part = s % 2
        soff = pl.multiple_of(j * STR + part * (NN // 2), 8)
        doff = pl.multiple_of((c * HPC + j) * NN + part * (NN // 2), 8)
        for g in range(2):
            wsh = wsh0 if g == 0 else wsh1
            wout = w1 if g == 0 else w2
            pltpu.sync_copy(wsh.at[pl.ds(soff, NN // 2)],
                            stg0.at[pl.ds(0, NN // 2)])
            pltpu.sync_copy(stg0.at[pl.ds(0, NN // 2)],
                            wout.at[pl.ds(doff, NN // 2)])


def _tc_epilogue(x1, x2, W1, W2, Wl, b1f, b2f, blf, w1, w2, o1, o2):
    rowh = lax.broadcasted_iota(jnp.int32, (H, H * D), 0)
    colh = lax.broadcasted_iota(jnp.int32, (H, H * D), 1) // D
    means = []
    for x, W, bf, wv in ((x1, W1, b1f, w1), (x2, W2, b2f, w2)):
        u = lax.dot_general(wv[...], x[...], (((1,), (0,)), ((), ())),
                            preferred_element_type=jnp.float32)
        P = jnp.dot(u, W[...], preferred_element_type=jnp.float32)
        msel = jnp.where(rowh == colh, P, 0.0)
        mean_flat = jnp.sum(msel, axis=0, keepdims=True) / NN + bf[...]
        means.append(mean_flat)
    o1[...] = jnp.dot(means[1], Wl[...],
                      preferred_element_type=jnp.float32) + blf[...]
    o2[...] = jnp.dot(means[0], Wl[...],
                      preferred_element_type=jnp.float32) + blf[...]


def _run_tc_epilogue(x1, x2, W1, W2, Wl, b1f, b2f, blf, w1, w2):
    shp = jax.ShapeDtypeStruct((1, 128), jnp.float32)
    return pl.pallas_call(
        _tc_epilogue,
        out_shape=[shp, shp],
    )(x1, x2, W1, W2, Wl, b1f, b2f, blf, w1, w2)


def kernel(x1, x2, edge_index1, edge_index2, W1, a_src1, a_dst1, b1,
           W2, a_src2, a_dst2, b2, Wl, bl):
    x1 = x1.astype(jnp.float32)
    x2 = x2.astype(jnp.float32)
    src1 = edge_index1[0].astype(jnp.int32)
    dst1 = edge_index1[1].astype(jnp.int32)
    src2 = edge_index2[0].astype(jnp.int32)
    dst2 = edge_index2[1].astype(jnp.int32)

    als, ald, ini = _run_tc_prologue(
        x1, x2, W1, a_src1.reshape(1, H * D), a_dst1.reshape(1, H * D),
        W2, a_src2.reshape(1, H * D), a_dst2.reshape(1, H * D))
    als1 = als[0].reshape(FLAT)
    ald1 = ald[0].reshape(FLAT)
    als2 = als[1].reshape(FLAT)
    ald2 = ald[1].reshape(FLAT)
    inip = jnp.pad(ini, ((0, 0), (0, 0), (0, STR - NN))).reshape(2, FPAD)

    w1, w2 = _sc_gat(src1, dst1, src2, dst2, als1, ald1, als2, ald2,
                     inip[0], inip[1])

    o1, o2 = _run_tc_epilogue(
        x1, x2, W1, W2, Wl,
        b1.reshape(1, H * D), b2.reshape(1, H * D), bl.reshape(1, 128),
        w1.reshape(H, NN), w2.reshape(H, NN))
    return (o1.reshape(128), o2.reshape(128))
